# baseline (speedup 1.0000x reference)
import functools

import jax
import jax.numpy as jnp
from jax import lax
from jax.experimental import pallas as pl
from jax.experimental.pallas import tpu as pltpu
from jax.experimental.pallas import tpu_sc as plsc

DIM = 128
B = 4096 * 50
NW = 32
B_PER_W = B // NW
CHUNK = 128
N_CHUNKS = B_PER_W // CHUNK
NBUF = 5

_mesh = plsc.VectorSubcoreMesh(core_axis_name="c", subcore_axis_name="s")


@functools.partial(
    pl.kernel,
    mesh=_mesh,
    out_type=jax.ShapeDtypeStruct((B, DIM), jnp.float32),
    scratch_types=[
        pltpu.VMEM((B_PER_W,), jnp.int32),
    ]
    + [pltpu.VMEM((CHUNK, DIM), jnp.float32)] * NBUF
    + [pltpu.SemaphoreType.DMA] * (2 * NBUF),
)
def _emb_lookup(idx_hbm, weight_hbm, out_hbm, idx_v, *scratch):
    bufs = scratch[:NBUF]
    gsems = scratch[NBUF : 2 * NBUF]
    ssems = scratch[2 * NBUF :]

    wid = lax.axis_index("s") * 2 + lax.axis_index("c")
    base = wid * B_PER_W
    pltpu.sync_copy(idx_hbm.at[pl.ds(base, B_PER_W)], idx_v)

    def gather_desc(i, b):
        return pltpu.make_async_copy(
            weight_hbm.at[idx_v.at[pl.ds(i * CHUNK, CHUNK)]], bufs[b], gsems[b]
        )

    def scatter_desc(i, b):
        return pltpu.make_async_copy(
            bufs[b], out_hbm.at[pl.ds(base + i * CHUNK, CHUNK)], ssems[b]
        )

    for p in range(NBUF - 1):
        gather_desc(p, p).start()

    def body(j, carry):
        for b in range(NBUF):
            i = NBUF * j + b
            nb = (b + NBUF - 1) % NBUF

            @pl.when(i >= 1)
            def _():
                scatter_desc(i - 1, nb).wait()

            @pl.when(i + NBUF - 1 < N_CHUNKS)
            def _():
                gather_desc(i + NBUF - 1, nb).start()

            gather_desc(i, b).wait()
            scatter_desc(i, b).start()
        return carry

    lax.fori_loop(0, N_CHUNKS // NBUF, body, 0)
    scatter_desc(N_CHUNKS - 1, (N_CHUNKS - 1) % NBUF).wait()


def kernel(x, weight):
    flat_idx = x.reshape(-1).astype(jnp.int32)
    out = _emb_lookup(flat_idx, weight)
    return out.reshape(x.shape + (DIM,))


# speedup vs baseline: 1.7893x; 1.7893x over previous
import functools

import jax
import jax.numpy as jnp
from jax import lax
from jax.experimental import pallas as pl
from jax.experimental.pallas import tpu as pltpu
from jax.experimental.pallas import tpu_sc as plsc

BATCH = 4096
SEQ = 50
DIM = 128
NW = 32
ROWS_PER_W = BATCH // NW
NBUF = 4

_mesh = plsc.VectorSubcoreMesh(core_axis_name="c", subcore_axis_name="s")


@functools.partial(
    pl.kernel,
    mesh=_mesh,
    out_type=jax.ShapeDtypeStruct((BATCH, SEQ, DIM), jnp.float32),
    scratch_types=[
        pltpu.VMEM((ROWS_PER_W, SEQ), jnp.int32),
    ]
    + [pltpu.VMEM((SEQ, DIM), jnp.float32)] * NBUF
    + [pltpu.SemaphoreType.DMA] * (2 * NBUF),
)
def _emb_lookup(idx_hbm, weight_hbm, out_hbm, idx_v, *scratch):
    bufs = scratch[:NBUF]
    gsems = scratch[NBUF : 2 * NBUF]
    ssems = scratch[2 * NBUF :]

    wid = lax.axis_index("s") * 2 + lax.axis_index("c")
    base = wid * ROWS_PER_W
    pltpu.sync_copy(idx_hbm.at[pl.ds(base, ROWS_PER_W)], idx_v)

    def gather_desc(i, b):
        return pltpu.make_async_copy(
            weight_hbm.at[idx_v.at[i]], bufs[b], gsems[b]
        )

    def scatter_desc(i, b):
        return pltpu.make_async_copy(bufs[b], out_hbm.at[base + i], ssems[b])

    for p in range(NBUF - 1):
        gather_desc(p, p).start()

    def body(j, carry):
        for b in range(NBUF):
            i = NBUF * j + b
            nb = (b + NBUF - 1) % NBUF

            @pl.when(i >= 1)
            def _():
                scatter_desc(i - 1, nb).wait()

            @pl.when(i + NBUF - 1 < ROWS_PER_W)
            def _():
                gather_desc(i + NBUF - 1, nb).start()

            gather_desc(i, b).wait()
            scatter_desc(i, b).start()
        return carry

    lax.fori_loop(0, ROWS_PER_W // NBUF, body, 0)
    scatter_desc(ROWS_PER_W - 1, (ROWS_PER_W - 1) % NBUF).wait()


def kernel(x, weight):
    return _emb_lookup(x.astype(jnp.int32), weight)


# speedup vs baseline: 1.7913x; 1.0011x over previous
import functools

import jax
import jax.numpy as jnp
from jax import lax
from jax.experimental import pallas as pl
from jax.experimental.pallas import tpu as pltpu
from jax.experimental.pallas import tpu_sc as plsc

BATCH = 4096
SEQ = 50
DIM = 128
NW = 32
ROWS_PER_W = BATCH // NW
NBUF = 4

_mesh = plsc.VectorSubcoreMesh(core_axis_name="c", subcore_axis_name="s")


@functools.partial(
    pl.kernel,
    mesh=_mesh,
    out_type=jax.ShapeDtypeStruct((BATCH, SEQ, DIM), jnp.float32),
    compiler_params=pltpu.CompilerParams(use_tc_tiling_on_sc=True),
    scratch_types=[
        pltpu.VMEM((ROWS_PER_W, SEQ), jnp.int32),
    ]
    + [pltpu.VMEM((SEQ, DIM), jnp.float32)] * NBUF
    + [pltpu.SemaphoreType.DMA] * (2 * NBUF),
)
def _emb_lookup(idx_hbm, weight_hbm, out_hbm, idx_v, *scratch):
    bufs = scratch[:NBUF]
    gsems = scratch[NBUF : 2 * NBUF]
    ssems = scratch[2 * NBUF :]

    wid = lax.axis_index("s") * 2 + lax.axis_index("c")
    base = wid * ROWS_PER_W
    pltpu.sync_copy(idx_hbm.at[pl.ds(base, ROWS_PER_W)], idx_v)

    def gather_desc(i, b):
        return pltpu.make_async_copy(
            weight_hbm.at[idx_v.at[i]], bufs[b], gsems[b]
        )

    def scatter_desc(i, b):
        return pltpu.make_async_copy(bufs[b], out_hbm.at[base + i], ssems[b])

    for p in range(NBUF - 1):
        gather_desc(p, p).start()

    def body(j, carry):
        for b in range(NBUF):
            i = NBUF * j + b
            nb = (b + NBUF - 1) % NBUF

            @pl.when(i >= 1)
            def _():
                scatter_desc(i - 1, nb).wait()

            @pl.when(i + NBUF - 1 < ROWS_PER_W)
            def _():
                gather_desc(i + NBUF - 1, nb).start()

            gather_desc(i, b).wait()
            scatter_desc(i, b).start()
        return carry

    lax.fori_loop(0, ROWS_PER_W // NBUF, body, 0)
    scatter_desc(ROWS_PER_W - 1, (ROWS_PER_W - 1) % NBUF).wait()


def kernel(x, weight):
    return _emb_lookup(x.astype(jnp.int32), weight)


# speedup vs baseline: 3.2133x; 1.7939x over previous
import functools

import jax
import jax.numpy as jnp
from jax import lax
from jax.experimental import pallas as pl
from jax.experimental.pallas import tpu as pltpu
from jax.experimental.pallas import tpu_sc as plsc

BATCH = 4096
SEQ = 50
DIM = 128
NW = 32
COLS = BATCH // NW
HALF = 64
NCH = SEQ * 2
NBUF = 12

_mesh = plsc.VectorSubcoreMesh(core_axis_name="c", subcore_axis_name="s")


@functools.partial(
    pl.kernel,
    mesh=_mesh,
    out_type=jax.ShapeDtypeStruct((SEQ, BATCH, DIM), jnp.float32),
    scratch_types=[
        pltpu.VMEM((SEQ, COLS), jnp.int32),
    ]
    + [pltpu.VMEM((HALF, DIM), jnp.float32)] * NBUF
    + [pltpu.SemaphoreType.DMA] * (2 * NBUF + 1),
)
def _emb_lookup(xt_hbm, weight_hbm, out_hbm, idx_v, *scratch):
    bufs = scratch[:NBUF]
    gsems = scratch[NBUF : 2 * NBUF]
    ssems = scratch[2 * NBUF : 3 * NBUF]
    isem = scratch[3 * NBUF]

    wid = lax.axis_index("s") * 2 + lax.axis_index("c")
    c0 = wid * COLS

    pltpu.async_copy(xt_hbm.at[:, pl.ds(c0, COLS)], idx_v, isem).wait()

    def gather_desc(i, b):
        s = i // 2
        off = (i % 2) * HALF
        return pltpu.make_async_copy(
            weight_hbm.at[idx_v.at[s, pl.ds(off, HALF)]], bufs[b], gsems[b]
        )

    def scatter_desc(i, b):
        s = i // 2
        off = (i % 2) * HALF
        return pltpu.make_async_copy(
            bufs[b], out_hbm.at[s, pl.ds(c0 + off, HALF)], ssems[b]
        )

    for p in range(NBUF - 1):
        gather_desc(p, p).start()

    def body(j, carry):
        for b in range(NBUF):
            i = NBUF * j + b
            nb = (b + NBUF - 1) % NBUF

            @pl.when(i >= 1)
            def _():
                scatter_desc(i - 1, nb).wait()

            @pl.when(i + NBUF - 1 < NCH)
            def _():
                gather_desc(i + NBUF - 1, nb).start()

            gather_desc(i, b).wait()
            scatter_desc(i, b).start()
        return carry

    lax.fori_loop(0, NCH // NBUF, body, 0)
    for i in range(NCH - NCH % NBUF, NCH):
        b = i % NBUF
        nb = (b + NBUF - 1) % NBUF
        scatter_desc(i - 1, nb).wait()
        if i + NBUF - 1 < NCH:
            gather_desc(i + NBUF - 1, nb).start()
        gather_desc(i, b).wait()
        scatter_desc(i, b).start()
    scatter_desc(NCH - 1, (NCH - 1) % NBUF).wait()


def kernel(x, weight):
    xt = jnp.swapaxes(x, 0, 1).astype(jnp.int32)
    out_t = _emb_lookup(xt, weight)
    return jnp.swapaxes(out_t, 0, 1)


# speedup vs baseline: 3.2558x; 1.0132x over previous
import functools

import jax
import jax.numpy as jnp
from jax import lax
from jax.experimental import pallas as pl
from jax.experimental.pallas import tpu as pltpu
from jax.experimental.pallas import tpu_sc as plsc

BATCH = 4096
SEQ = 50
DIM = 128
NW = 32
COLS = BATCH // NW
NBUF = 7

_mesh = plsc.VectorSubcoreMesh(core_axis_name="c", subcore_axis_name="s")


@functools.partial(
    pl.kernel,
    mesh=_mesh,
    out_type=jax.ShapeDtypeStruct((SEQ, BATCH, DIM), jnp.float32),
    scratch_types=[
        pltpu.VMEM((SEQ, COLS), jnp.int32),
    ]
    + [pltpu.VMEM((COLS, DIM), jnp.float32)] * NBUF
    + [pltpu.SemaphoreType.DMA] * (2 * NBUF + 1),
)
def _emb_lookup(xt_hbm, weight_hbm, out_hbm, idx_v, *scratch):
    bufs = scratch[:NBUF]
    gsems = scratch[NBUF : 2 * NBUF]
    ssems = scratch[2 * NBUF : 3 * NBUF]
    isem = scratch[3 * NBUF]

    wid = lax.axis_index("s") * 2 + lax.axis_index("c")
    c0 = wid * COLS

    pltpu.async_copy(xt_hbm.at[:, pl.ds(c0, COLS)], idx_v, isem).wait()

    def gather_desc(i, b):
        return pltpu.make_async_copy(
            weight_hbm.at[idx_v.at[i]], bufs[b], gsems[b]
        )

    def scatter_desc(i, b):
        return pltpu.make_async_copy(
            bufs[b], out_hbm.at[i, pl.ds(c0, COLS)], ssems[b]
        )

    for p in range(NBUF - 1):
        gather_desc(p, p).start()

    def body(j, carry):
        for b in range(NBUF):
            i = NBUF * j + b
            nb = (b + NBUF - 1) % NBUF

            @pl.when(i >= 1)
            def _():
                scatter_desc(i - 1, nb).wait()

            @pl.when(i + NBUF - 1 < SEQ)
            def _():
                gather_desc(i + NBUF - 1, nb).start()

            gather_desc(i, b).wait()
            scatter_desc(i, b).start()
        return carry

    lax.fori_loop(0, SEQ // NBUF, body, 0)
    for i in range(SEQ - SEQ % NBUF, SEQ):
        b = i % NBUF
        nb = (b + NBUF - 1) % NBUF
        scatter_desc(i - 1, nb).wait()
        if i + NBUF - 1 < SEQ:
            gather_desc(i + NBUF - 1, nb).start()
        gather_desc(i, b).wait()
        scatter_desc(i, b).start()
    scatter_desc(SEQ - 1, (SEQ - 1) % NBUF).wait()


def kernel(x, weight):
    xt = jnp.swapaxes(x, 0, 1).astype(jnp.int32)
    out_t = _emb_lookup(xt, weight)
    return jnp.swapaxes(out_t, 0, 1)
